# Initial kernel scaffold; baseline (speedup 1.0000x reference)
import jax
import jax.numpy as jnp
from jax.experimental import pallas as pl


def kernel(x, integration_weight, mu, Wr1, br1, Wr2, br2, Wh1, bh1, Wh2, bh2, expert_w1, expert_b1, expert_w2, expert_b2, Ws1, bs1, Ws2, bs2, shared_weight, Wf1, bf1, Wf2, bf2):
    raise NotImplementedError("write your pallas kernel here")


# speedup vs baseline: 3.2966x; 3.2966x over previous
import jax
import jax.numpy as jnp
from jax.experimental import pallas as pl
from jax.experimental.pallas import tpu as pltpu

D = 768
E = 8
TOP_K = 2
NUM_ITER = 2
DT = 0.1
H = 64
CTX = 2 * D
TB = 256


def _body(x_ref, iw_ref, mu_ref, Wr1_ref, br1_ref, Wr2_ref, br2_ref,
          Wh1_ref, bh1_ref, wh2_ref, bh2_ref,
          W1a_ref, W1b_ref, b1_ref, W2_ref, b2e_ref,
          Ws1a_ref, Ws1b_ref, bs1_ref, Ws2_ref, bs2_ref, swt_ref,
          Wf1_ref, bf1_ref, Wf2_ref, bf2_ref, out_ref):
    f32 = jnp.float32
    tokens = x_ref[...]
    tb = tokens.shape[0]

    r1 = jax.nn.gelu(jnp.dot(tokens, Wr1_ref[...],
                             preferred_element_type=f32) + br1_ref[...])
    logits = jnp.dot(r1, Wr2_ref[...], preferred_element_type=f32) + br2_ref[...]
    probs = jax.nn.softmax(logits, axis=-1)

    iota_e = jax.lax.broadcasted_iota(jnp.int32, (tb, E), 1)
    m1 = jnp.max(probs, axis=1, keepdims=True)
    i1 = jnp.min(jnp.where(probs == m1, iota_e, E), axis=1, keepdims=True)
    sel1 = iota_e == i1
    probs_m = jnp.where(sel1, -jnp.inf, probs)
    m2 = jnp.max(probs_m, axis=1, keepdims=True)
    i2 = jnp.min(jnp.where(probs_m == m2, iota_e, E), axis=1, keepdims=True)
    sel2 = iota_e == i2
    denom = m1 + m2
    combine = (jnp.where(sel1, m1, 0.0) + jnp.where(sel2, m2, 0.0)) / denom

    row_e = jax.lax.broadcasted_iota(jnp.int32, (E, E * H), 0)
    col_e = jax.lax.broadcasted_iota(jnp.int32, (E, E * H), 1) // H
    expand = (row_e == col_e).astype(f32)
    comb_h = jnp.dot(combine, expand, preferred_element_type=f32)

    mu = mu_ref[...]
    swt = swt_ref[0, 0]

    xs = tokens
    v = jnp.zeros_like(tokens)
    for _ in range(NUM_ITER):
        h = jax.nn.gelu(
            jnp.dot(xs, W1a_ref[...], preferred_element_type=f32)
            + jnp.dot(v, W1b_ref[...], preferred_element_type=f32)
            + b1_ref[...])
        ctrl = (jnp.dot(h * comb_h, W2_ref[...], preferred_element_type=f32)
                + jnp.dot(combine, b2e_ref[...], preferred_element_type=f32))
        sh = jax.nn.gelu(
            jnp.dot(xs, Ws1a_ref[...], preferred_element_type=f32)
            + jnp.dot(v, Ws1b_ref[...], preferred_element_type=f32)
            + bs1_ref[...])
        shared = jnp.dot(sh, Ws2_ref[...], preferred_element_type=f32) + bs2_ref[...]
        ctrl = ctrl + swt * shared
        alpha = jax.nn.sigmoid(ctrl[:, :D])
        beta = jax.nn.softplus(ctrl[:, D:2 * D])
        gate = jax.nn.sigmoid(ctrl[:, 2 * D:])
        err = xs - mu
        v = alpha * v - beta * err
        xs = xs + DT * gate * v

    hh = jax.nn.gelu(jnp.dot(xs, Wh1_ref[...], preferred_element_type=f32)
                     + bh1_ref[...])
    halt = jax.nn.sigmoid(jnp.sum(hh * wh2_ref[...], axis=1, keepdims=True)
                          + bh2_ref[...])
    rf = jax.nn.gelu(jnp.dot(xs, Wf1_ref[...], preferred_element_type=f32)
                     + bf1_ref[...])
    refined = jnp.dot(rf, Wf2_ref[...], preferred_element_type=f32) + bf2_ref[...]
    out_ref[...] = tokens + iw_ref[...] * (halt * refined)


def kernel(x, integration_weight, mu, Wr1, br1, Wr2, br2, Wh1, bh1, Wh2, bh2,
           expert_w1, expert_b1, expert_w2, expert_b2,
           Ws1, bs1, Ws2, bs2, shared_weight, Wf1, bf1, Wf2, bf2):
    B, N, Dd = x.shape
    T = B * N
    f32 = jnp.float32
    xt = x.reshape(T, Dd)

    W1a = expert_w1[:, :D, :].transpose(1, 0, 2).reshape(D, E * H)
    W1b = expert_w1[:, D:, :].transpose(1, 0, 2).reshape(D, E * H)
    b1 = expert_b1.reshape(1, E * H)
    W2 = expert_w2.reshape(E * H, 3 * D)

    row2 = lambda a: a.reshape(1, -1)
    ops = (xt, row2(integration_weight), row2(mu), Wr1, row2(br1), Wr2,
           row2(br2), Wh1, row2(bh1), Wh2.reshape(1, -1),
           jnp.asarray(bh2, f32).reshape(1, 1),
           W1a, W1b, b1, W2, expert_b2,
           Ws1[:D], Ws1[D:], row2(bs1), Ws2, row2(bs2),
           jnp.asarray(shared_weight, f32).reshape(1, 1),
           Wf1, row2(bf1), Wf2, row2(bf2))

    full = lambda a: pl.BlockSpec(a.shape, lambda i: (0,) * a.ndim)
    in_specs = [pl.BlockSpec((TB, Dd), lambda i: (i, 0))]
    in_specs += [full(a) for a in ops[1:]]

    out = pl.pallas_call(
        _body,
        grid=(T // TB,),
        in_specs=in_specs,
        out_specs=pl.BlockSpec((TB, Dd), lambda i: (i, 0)),
        out_shape=jax.ShapeDtypeStruct((T, Dd), f32),
        compiler_params=pltpu.CompilerParams(
            dimension_semantics=("arbitrary",)),
    )(*ops)
    return out.reshape(B, N, Dd)
